# baseline (speedup 1.0000x reference)
import functools

import jax
import jax.numpy as jnp
from jax import lax
from jax.experimental import pallas as pl
from jax.experimental.pallas import tpu as pltpu
from jax.experimental.pallas import tpu_sc as plsc

F32 = jnp.float32
LAT = 128
EOUT = 16
NUM_CORE_STEPS = 5

NCORES = 2
NSUB = 16
NWORK = NCORES * NSUB
CH = 80
CHS = 2000


def _node_proj_body(x_ref, wsrc_ref, wdst_ref, u_ref, wu_ref, b1_ref,
                    xs_ref, xd_ref):
    x = x_ref[...]
    ub = jnp.dot(u_ref[...], wu_ref[...], preferred_element_type=F32) + b1_ref[...]
    xs = jnp.dot(x, wsrc_ref[...], preferred_element_type=F32) + ub
    xd = jnp.dot(x, wdst_ref[...], preferred_element_type=F32)
    xs_ref[...] = xs.astype(jnp.bfloat16)
    xd_ref[...] = xd.astype(jnp.bfloat16)


def _node_proj(x, wsrc, wdst, u, wu, b1):
    n, dx = x.shape
    du = u.shape[1]
    bn = 1264
    grid = n // bn
    return pl.pallas_call(
        _node_proj_body,
        grid=(grid,),
        in_specs=[
            pl.BlockSpec((bn, dx), lambda i: (i, 0)),
            pl.BlockSpec((dx, LAT), lambda i: (0, 0)),
            pl.BlockSpec((dx, LAT), lambda i: (0, 0)),
            pl.BlockSpec((1, du), lambda i: (0, 0)),
            pl.BlockSpec((du, LAT), lambda i: (0, 0)),
            pl.BlockSpec((1, LAT), lambda i: (0, 0)),
        ],
        out_specs=[pl.BlockSpec((bn, LAT), lambda i: (i, 0)),
                   pl.BlockSpec((bn, LAT), lambda i: (i, 0))],
        out_shape=[jax.ShapeDtypeStruct((n, LAT), jnp.bfloat16),
                   jax.ShapeDtypeStruct((n, LAT), jnp.bfloat16)],
    )(x, wsrc, wdst, u, wu, b1)


def _edge_mlp_body(gr_ref, gc_ref, ef_ref, wef_ref, w2_ref, b2_ref,
                   e_ref, et_ref):
    h = (gr_ref[...].astype(F32) + gc_ref[...].astype(F32)
         + jnp.dot(ef_ref[...], wef_ref[...], preferred_element_type=F32))
    h = jnp.maximum(h, 0.0)
    e = jnp.dot(h, w2_ref[...], preferred_element_type=F32) + b2_ref[...]
    e_ref[...] = e
    et_ref[...] = jnp.transpose(e)[None]


def _edge_mlp(gr, gc, efeat, wef, w2, b2):
    e_num, de = efeat.shape
    be = CHS
    grid = e_num // be
    return pl.pallas_call(
        _edge_mlp_body,
        grid=(grid,),
        in_specs=[
            pl.BlockSpec((be, LAT), lambda i: (i, 0)),
            pl.BlockSpec((be, LAT), lambda i: (i, 0)),
            pl.BlockSpec((be, de), lambda i: (i, 0)),
            pl.BlockSpec((de, LAT), lambda i: (0, 0)),
            pl.BlockSpec((LAT, EOUT), lambda i: (0, 0)),
            pl.BlockSpec((1, EOUT), lambda i: (0, 0)),
        ],
        out_specs=[pl.BlockSpec((be, EOUT), lambda i: (i, 0)),
                   pl.BlockSpec((1, EOUT, be), lambda i: (i, 0, 0))],
        out_shape=[jax.ShapeDtypeStruct((e_num, EOUT), F32),
                   jax.ShapeDtypeStruct((e_num // be, EOUT, be), F32)],
    )(gr, gc, efeat, wef, w2, b2)


def _node_mlp_body(n_real, x_ref, a0_ref, a1_ref, u_ref,
                   wnx_ref, wna_ref, wnu_ref, b1n_ref, w2n_ref, b2n_ref,
                   wgm_ref, wgu_ref, b1g_ref, w2g_ref, b2g_ref,
                   xn_ref, gu_ref):
    at = a0_ref[...] + a1_ref[...]
    pre = (jnp.dot(x_ref[...], wnx_ref[...], preferred_element_type=F32)
           + jax.lax.dot_general(at, wna_ref[...], (((0,), (0,)), ((), ())),
                                 preferred_element_type=F32)
           + jnp.dot(u_ref[...], wnu_ref[...], preferred_element_type=F32)
           + b1n_ref[...])
    xn = (jnp.dot(jnp.maximum(pre, 0.0), w2n_ref[...],
                  preferred_element_type=F32) + b2n_ref[...])
    rows = jax.lax.broadcasted_iota(jnp.int32, xn.shape, 0)
    xn = jnp.where(rows < n_real, xn, 0.0)
    xn_ref[...] = xn
    mean = jnp.sum(xn, axis=0, keepdims=True) * (1.0 / n_real)
    hg = jnp.maximum(
        jnp.dot(mean, wgm_ref[...], preferred_element_type=F32)
        + jnp.dot(u_ref[...], wgu_ref[...], preferred_element_type=F32)
        + b1g_ref[...], 0.0)
    gu_ref[...] = (jnp.dot(hg, w2g_ref[...], preferred_element_type=F32)
                   + b2g_ref[...])


def _node_mlp(x, a0, a1, u, wnx, wna, wnu, b1n, w2n, b2n,
              wgm, wgu, b1g, w2g, b2g, n_real):
    n, dx = x.shape
    du = u.shape[1]
    dn = w2n.shape[1]
    dg = w2g.shape[1]
    return pl.pallas_call(
        functools.partial(_node_mlp_body, n_real),
        out_shape=[jax.ShapeDtypeStruct((n, dn), F32),
                   jax.ShapeDtypeStruct((1, dg), F32)],
    )(x, a0, a1, u, wnx, wna, wnu, b1n, w2n, b2n, wgm, wgu, b1g, w2g, b2g)


def _edge_gather(xs, xd, row2, col2):
    wds = LAT // 2
    e_num = row2.shape[0]
    ew = e_num // NWORK
    nsc = 5
    bat = nsc * CH
    nbt = ew // bat
    mesh = plsc.VectorSubcoreMesh(core_axis_name="c", subcore_axis_name="s")

    @functools.partial(
        pl.kernel, mesh=mesh,
        out_type=[jax.ShapeDtypeStruct((e_num, wds), jnp.int32),
                  jax.ShapeDtypeStruct((e_num, wds), jnp.int32)],
        compiler_params=pltpu.CompilerParams(use_tc_tiling_on_sc=False),
        scratch_types=[
            pltpu.VMEM((bat,), jnp.int32), pltpu.VMEM((bat,), jnp.int32),
            pltpu.VMEM((bat, wds), jnp.int32), pltpu.VMEM((bat, wds), jnp.int32),
            pltpu.SemaphoreType.DMA, pltpu.SemaphoreType.DMA,
        ],
    )
    def gather_k(xs_hbm, xd_hbm, row_hbm, col_hbm, gr_hbm, gc_hbm,
                 ridx, cidx, bufr, bufc, sem_r, sem_c):
        wid = lax.axis_index("s") * NCORES + lax.axis_index("c")

        def body(j, carry):
            base = wid * ew + j * bat
            pltpu.sync_copy(row_hbm.at[pl.ds(base, bat)], ridx)
            pltpu.sync_copy(col_hbm.at[pl.ds(base, bat)], cidx)
            cps = []
            for k in range(nsc):
                cps.append(pltpu.async_copy(
                    xs_hbm.at[ridx.at[pl.ds(k * CH, CH)]],
                    bufr.at[pl.ds(k * CH, CH)], sem_r))
                cps.append(pltpu.async_copy(
                    xd_hbm.at[cidx.at[pl.ds(k * CH, CH)]],
                    bufc.at[pl.ds(k * CH, CH)], sem_c))
            for cp in cps:
                cp.wait()
            ebase = wid * ew + j * bat
            pltpu.sync_copy(bufr, gr_hbm.at[pl.ds(ebase, bat)])
            pltpu.sync_copy(bufc, gc_hbm.at[pl.ds(ebase, bat)])
            return carry

        lax.fori_loop(0, nbt, body, 0)

    return gather_k(xs, xd, row2, col2)


def _edge_scatter(etc, col, n_pad):
    e_num = col.shape[0]
    rh = n_pad // 2
    acc_sz = EOUT * rh
    ec = e_num // NCORES
    ew2 = ec // (NSUB // 2)
    nck = ew2 // CHS
    mesh = plsc.VectorSubcoreMesh(core_axis_name="c", subcore_axis_name="s")

    @functools.partial(
        pl.kernel, mesh=mesh,
        out_type=jax.ShapeDtypeStruct((NCORES * EOUT * n_pad,), F32),
        compiler_params=pltpu.CompilerParams(needs_layout_passes=False,
                                             use_tc_tiling_on_sc=False),
        scratch_types=[
            pltpu.VMEM((CHS,), jnp.int32),
            pltpu.VMEM((EOUT, CHS), F32),
            pltpu.VMEM((EOUT, rh), F32),
            pltpu.HBM((NCORES, NSUB, acc_sz), F32),
        ],
    )
    def scatter_k(etc_hbm, col_hbm, out_hbm, cidx, ebt, acc, stage):
        cid = lax.axis_index("c")
        sid = lax.axis_index("s")
        half = sid // (NSUB // 2)
        grp = sid % (NSUB // 2)
        nbase = half * rh

        def zero_body(k, carry):
            for f in range(EOUT):
                acc[f, pl.ds(k * 16, 16)] = jnp.zeros((16,), F32)
            return carry

        lax.fori_loop(0, rh // 16, zero_body, 0)

        def chunk_body(c, carry):
            ebase = cid * ec + grp * ew2 + c * CHS
            pltpu.sync_copy(col_hbm.at[pl.ds(ebase, CHS)], cidx)
            pltpu.sync_copy(etc_hbm.at[ebase // CHS], ebt)

            def j_body(j, carry2):
                idx = cidx[pl.ds(j * 16, 16)]
                idxm = idx - nbase
                mask = (idxm >= 0) & (idxm < rh)
                for f in range(EOUT):
                    vals = ebt[f, pl.ds(j * 16, 16)]
                    frow = jnp.full((16,), f, jnp.int32)
                    plsc.addupdate_scatter(acc, [frow, idxm], vals, mask=mask)
                return carry2

            lax.fori_loop(0, CHS // 16, j_body, 0)
            return carry

        lax.fori_loop(0, nck, chunk_body, 0)

        for f in range(EOUT):
            pltpu.sync_copy(acc.at[f],
                            stage.at[cid, sid, pl.ds(f * rh, rh)])
        plsc.subcore_barrier()

        for t in range(2):
            task = sid * 2 + t
            h2 = task // EOUT
            f = task % EOUT

            def load_body(g, carry):
                pltpu.sync_copy(
                    stage.at[cid, h2 * 8 + g, pl.ds(f * rh, rh)],
                    acc.at[g])
                return carry

            lax.fori_loop(0, 8, load_body, 0)

            def red_body(k, carry):
                v = acc[0, pl.ds(k * 16, 16)]
                for g in range(1, 8):
                    v = v + acc[g, pl.ds(k * 16, 16)]
                acc[0, pl.ds(k * 16, 16)] = v
                return carry

            lax.fori_loop(0, rh // 16, red_body, 0)
            pltpu.sync_copy(
                acc.at[0],
                out_hbm.at[pl.ds(cid * EOUT * n_pad + f * n_pad + h2 * rh,
                                 rh)])

    return scatter_k(etc, col)


def _meta_step(p, x, efeat, u, row2, col2, col, n_real):
    n = x.shape[0]
    dx = x.shape[1]
    de = efeat.shape[1]
    pe, pn, pg = p["edge"], p["node"], p["global"]

    w1e = pe["w1"]
    wsrc, wdst = w1e[:dx], w1e[dx:2 * dx]
    wef, wue = w1e[2 * dx:2 * dx + de], w1e[2 * dx + de:]
    xs, xd = _node_proj(x, wsrc, wdst, u, wue, pe["b1"].reshape(1, -1))

    xs_i = jax.lax.bitcast_convert_type(xs.reshape(n, LAT // 2, 2), jnp.int32)
    xd_i = jax.lax.bitcast_convert_type(xd.reshape(n, LAT // 2, 2), jnp.int32)
    gr_i, gc_i = _edge_gather(xs_i, xd_i, row2, col2)
    e_num = gr_i.shape[0]
    gr = jax.lax.bitcast_convert_type(gr_i, jnp.bfloat16).reshape(e_num, LAT)
    gc = jax.lax.bitcast_convert_type(gc_i, jnp.bfloat16).reshape(e_num, LAT)
    e, etc = _edge_mlp(gr, gc, efeat, wef, pe["w2"], pe["b2"].reshape(1, -1))

    agg = _edge_scatter(etc, col, n).reshape(NCORES, EOUT, n)

    w1n = pn["w1"]
    wnx, wna, wnu = w1n[:dx], w1n[dx:dx + EOUT], w1n[dx + EOUT:]
    w1g = pg["w1"]
    nout = pn["w2"].shape[1]
    wgm, wgu = w1g[:nout], w1g[nout:]
    xn, gu = _node_mlp(
        x, agg[0], agg[1], u,
        wnx, wna, wnu, pn["b1"].reshape(1, -1), pn["w2"], pn["b2"].reshape(1, -1),
        wgm, wgu, pg["b1"].reshape(1, -1), pg["w2"], pg["b2"].reshape(1, -1),
        n_real)
    return xn, e, gu


def kernel(x, edge_attr, global_attr, params, edge_index):
    row2, col2 = edge_index[0], edge_index[1]
    col = col2
    n_real = x.shape[0]
    n_pad = ((n_real + 127) // 128) * 128
    xp = jnp.pad(x, ((0, n_pad - n_real), (0, 0)))
    xc, ec, uc = _meta_step(params["encoder"], xp, edge_attr, global_attr,
                            row2, col2, col, n_real)
    x0, e0, u0 = xc, ec, uc
    for _ in range(NUM_CORE_STEPS):
        xc, ec, uc = _meta_step(
            params["core"],
            jnp.concatenate([x0, xc], axis=1),
            jnp.concatenate([e0, ec], axis=1),
            jnp.concatenate([u0, uc], axis=1),
            row2, col2, col, n_real)
    xn, e, gu = _meta_step(params["decoder"], xc, ec, uc, row2, col2, col,
                           n_real)
    return xn[:n_real], e, gu


# speedup vs baseline: 2.9516x; 2.9516x over previous
import functools

import jax
import jax.numpy as jnp
from jax import lax
from jax.experimental import pallas as pl
from jax.experimental.pallas import tpu as pltpu
from jax.experimental.pallas import tpu_sc as plsc

F32 = jnp.float32
LAT = 128
EOUT = 16
NUM_CORE_STEPS = 5

NCORES = 2
NSUB = 16
NWORK = NCORES * NSUB
CH = 80
CHS = 2000


def _node_proj_body(x_ref, wsrc_ref, wdst_ref, u_ref, wu_ref, b1_ref,
                    xs_ref, xd_ref):
    x = x_ref[...]
    ub = jnp.dot(u_ref[...], wu_ref[...], preferred_element_type=F32) + b1_ref[...]
    xs_ref[...] = jnp.dot(x, wsrc_ref[...], preferred_element_type=F32) + ub
    xd_ref[...] = jnp.dot(x, wdst_ref[...], preferred_element_type=F32)


def _node_proj(x, wsrc, wdst, u, wu, b1):
    n, dx = x.shape
    du = u.shape[1]
    bn = 1264
    grid = n // bn
    return pl.pallas_call(
        _node_proj_body,
        grid=(grid,),
        in_specs=[
            pl.BlockSpec((bn, dx), lambda i: (i, 0)),
            pl.BlockSpec((dx, LAT), lambda i: (0, 0)),
            pl.BlockSpec((dx, LAT), lambda i: (0, 0)),
            pl.BlockSpec((1, du), lambda i: (0, 0)),
            pl.BlockSpec((du, LAT), lambda i: (0, 0)),
            pl.BlockSpec((1, LAT), lambda i: (0, 0)),
        ],
        out_specs=[pl.BlockSpec((bn, LAT), lambda i: (i, 0)),
                   pl.BlockSpec((bn, LAT), lambda i: (i, 0))],
        out_shape=[jax.ShapeDtypeStruct((n, LAT), F32),
                   jax.ShapeDtypeStruct((n, LAT), F32)],
    )(x, wsrc, wdst, u, wu, b1)


def _edge_mlp_body(gr_ref, gc_ref, ef_ref, wef_ref, w2_ref, b2_ref,
                   e_ref, et_ref):
    h = (gr_ref[...].astype(F32) + gc_ref[...].astype(F32)
         + jnp.dot(ef_ref[...], wef_ref[...], preferred_element_type=F32))
    h = jnp.maximum(h, 0.0)
    e = jnp.dot(h, w2_ref[...], preferred_element_type=F32) + b2_ref[...]
    e_ref[...] = e
    et_ref[...] = jnp.transpose(e)[None]


def _edge_mlp(gr, gc, efeat, wef, w2, b2):
    e_num, de = efeat.shape
    be = CHS
    grid = e_num // be
    return pl.pallas_call(
        _edge_mlp_body,
        grid=(grid,),
        in_specs=[
            pl.BlockSpec((be, LAT), lambda i: (i, 0)),
            pl.BlockSpec((be, LAT), lambda i: (i, 0)),
            pl.BlockSpec((be, de), lambda i: (i, 0)),
            pl.BlockSpec((de, LAT), lambda i: (0, 0)),
            pl.BlockSpec((LAT, EOUT), lambda i: (0, 0)),
            pl.BlockSpec((1, EOUT), lambda i: (0, 0)),
        ],
        out_specs=[pl.BlockSpec((be, EOUT), lambda i: (i, 0)),
                   pl.BlockSpec((1, EOUT, be), lambda i: (i, 0, 0))],
        out_shape=[jax.ShapeDtypeStruct((e_num, EOUT), F32),
                   jax.ShapeDtypeStruct((e_num // be, EOUT, be), F32)],
    )(gr, gc, efeat, wef, w2, b2)


def _node_mlp_body(n_real, x_ref, a0_ref, a1_ref, u_ref,
                   wnx_ref, wna_ref, wnu_ref, b1n_ref, w2n_ref, b2n_ref,
                   wgm_ref, wgu_ref, b1g_ref, w2g_ref, b2g_ref,
                   xn_ref, gu_ref):
    at = a0_ref[...] + a1_ref[...]
    pre = (jnp.dot(x_ref[...], wnx_ref[...], preferred_element_type=F32)
           + jax.lax.dot_general(at, wna_ref[...], (((0,), (0,)), ((), ())),
                                 preferred_element_type=F32)
           + jnp.dot(u_ref[...], wnu_ref[...], preferred_element_type=F32)
           + b1n_ref[...])
    xn = (jnp.dot(jnp.maximum(pre, 0.0), w2n_ref[...],
                  preferred_element_type=F32) + b2n_ref[...])
    rows = jax.lax.broadcasted_iota(jnp.int32, xn.shape, 0)
    xn = jnp.where(rows < n_real, xn, 0.0)
    xn_ref[...] = xn
    mean = jnp.sum(xn, axis=0, keepdims=True) * (1.0 / n_real)
    hg = jnp.maximum(
        jnp.dot(mean, wgm_ref[...], preferred_element_type=F32)
        + jnp.dot(u_ref[...], wgu_ref[...], preferred_element_type=F32)
        + b1g_ref[...], 0.0)
    gu_ref[...] = (jnp.dot(hg, w2g_ref[...], preferred_element_type=F32)
                   + b2g_ref[...])


def _node_mlp(x, a0, a1, u, wnx, wna, wnu, b1n, w2n, b2n,
              wgm, wgu, b1g, w2g, b2g, n_real):
    n, dx = x.shape
    du = u.shape[1]
    dn = w2n.shape[1]
    dg = w2g.shape[1]
    return pl.pallas_call(
        functools.partial(_node_mlp_body, n_real),
        out_shape=[jax.ShapeDtypeStruct((n, dn), F32),
                   jax.ShapeDtypeStruct((1, dg), F32)],
    )(x, a0, a1, u, wnx, wna, wnu, b1n, w2n, b2n, wgm, wgu, b1g, w2g, b2g)


def _edge_gather(xs, xd, row2, col2):
    wds = LAT
    e_num = row2.shape[0]
    ew = e_num // NWORK
    nsc = 5
    bat = nsc * CH
    nbt = ew // bat
    mesh = plsc.VectorSubcoreMesh(core_axis_name="c", subcore_axis_name="s")

    @functools.partial(
        pl.kernel, mesh=mesh,
        out_type=[jax.ShapeDtypeStruct((e_num, wds), F32),
                  jax.ShapeDtypeStruct((e_num, wds), F32)],
        scratch_types=[
            pltpu.VMEM((bat,), jnp.int32), pltpu.VMEM((bat,), jnp.int32),
            pltpu.VMEM((bat, wds), F32), pltpu.VMEM((bat, wds), F32),
            pltpu.SemaphoreType.DMA, pltpu.SemaphoreType.DMA,
        ],
    )
    def gather_k(xs_hbm, xd_hbm, row_hbm, col_hbm, gr_hbm, gc_hbm,
                 ridx, cidx, bufr, bufc, sem_r, sem_c):
        wid = lax.axis_index("s") * NCORES + lax.axis_index("c")

        def body(j, carry):
            base = wid * ew + j * bat
            pltpu.sync_copy(row_hbm.at[pl.ds(base, bat)], ridx)
            pltpu.sync_copy(col_hbm.at[pl.ds(base, bat)], cidx)
            cps = []
            for k in range(nsc):
                cps.append(pltpu.async_copy(
                    xs_hbm.at[ridx.at[pl.ds(k * CH, CH)]],
                    bufr.at[pl.ds(k * CH, CH)], sem_r))
                cps.append(pltpu.async_copy(
                    xd_hbm.at[cidx.at[pl.ds(k * CH, CH)]],
                    bufc.at[pl.ds(k * CH, CH)], sem_c))
            for cp in cps:
                cp.wait()
            ebase = wid * ew + j * bat
            pltpu.sync_copy(bufr, gr_hbm.at[pl.ds(ebase, bat)])
            pltpu.sync_copy(bufc, gc_hbm.at[pl.ds(ebase, bat)])
            return carry

        lax.fori_loop(0, nbt, body, 0)

    return gather_k(xs, xd, row2, col2)


def _edge_scatter(etc, col, n_pad):
    e_num = col.shape[0]
    rh = n_pad // 2
    acc_sz = EOUT * rh
    ec = e_num // NCORES
    ew2 = ec // (NSUB // 2)
    nck = ew2 // CHS
    mesh = plsc.VectorSubcoreMesh(core_axis_name="c", subcore_axis_name="s")

    @functools.partial(
        pl.kernel, mesh=mesh,
        out_type=jax.ShapeDtypeStruct((NCORES * EOUT * n_pad,), F32),
        compiler_params=pltpu.CompilerParams(needs_layout_passes=False,
                                             use_tc_tiling_on_sc=False),
        scratch_types=[
            pltpu.VMEM((CHS,), jnp.int32),
            pltpu.VMEM((EOUT, CHS), F32),
            pltpu.VMEM((EOUT, rh), F32),
            pltpu.HBM((NCORES, NSUB, acc_sz), F32),
        ],
    )
    def scatter_k(etc_hbm, col_hbm, out_hbm, cidx, ebt, acc, stage):
        cid = lax.axis_index("c")
        sid = lax.axis_index("s")
        half = sid // (NSUB // 2)
        grp = sid % (NSUB // 2)
        nbase = half * rh

        def zero_body(k, carry):
            for f in range(EOUT):
                acc[f, pl.ds(k * 16, 16)] = jnp.zeros((16,), F32)
            return carry

        lax.fori_loop(0, rh // 16, zero_body, 0)

        def chunk_body(c, carry):
            ebase = cid * ec + grp * ew2 + c * CHS
            pltpu.sync_copy(col_hbm.at[pl.ds(ebase, CHS)], cidx)
            pltpu.sync_copy(etc_hbm.at[ebase // CHS], ebt)

            def j_body(j, carry2):
                idx = cidx[pl.ds(j * 16, 16)]
                idxm = idx - nbase
                mask = (idxm >= 0) & (idxm < rh)
                for f in range(EOUT):
                    vals = ebt[f, pl.ds(j * 16, 16)]
                    frow = jnp.full((16,), f, jnp.int32)
                    plsc.addupdate_scatter(acc, [frow, idxm], vals, mask=mask)
                return carry2

            lax.fori_loop(0, CHS // 16, j_body, 0)
            return carry

        lax.fori_loop(0, nck, chunk_body, 0)

        for f in range(EOUT):
            pltpu.sync_copy(acc.at[f],
                            stage.at[cid, sid, pl.ds(f * rh, rh)])
        plsc.subcore_barrier()

        for t in range(2):
            task = sid * 2 + t
            h2 = task // EOUT
            f = task % EOUT

            def load_body(g, carry):
                pltpu.sync_copy(
                    stage.at[cid, h2 * 8 + g, pl.ds(f * rh, rh)],
                    acc.at[g])
                return carry

            lax.fori_loop(0, 8, load_body, 0)

            def red_body(k, carry):
                v = acc[0, pl.ds(k * 16, 16)]
                for g in range(1, 8):
                    v = v + acc[g, pl.ds(k * 16, 16)]
                acc[0, pl.ds(k * 16, 16)] = v
                return carry

            lax.fori_loop(0, rh // 16, red_body, 0)
            pltpu.sync_copy(
                acc.at[0],
                out_hbm.at[pl.ds(cid * EOUT * n_pad + f * n_pad + h2 * rh,
                                 rh)])

    return scatter_k(etc, col)


def _meta_step(p, x, efeat, u, row2, col2, col, n_real):
    n = x.shape[0]
    dx = x.shape[1]
    de = efeat.shape[1]
    pe, pn, pg = p["edge"], p["node"], p["global"]

    w1e = pe["w1"]
    wsrc, wdst = w1e[:dx], w1e[dx:2 * dx]
    wef, wue = w1e[2 * dx:2 * dx + de], w1e[2 * dx + de:]
    xs, xd = _node_proj(x, wsrc, wdst, u, wue, pe["b1"].reshape(1, -1))

    gr, gc = _edge_gather(xs, xd, row2, col2)
    e, etc = _edge_mlp(gr, gc, efeat, wef, pe["w2"], pe["b2"].reshape(1, -1))

    agg = _edge_scatter(etc, col, n).reshape(NCORES, EOUT, n)

    w1n = pn["w1"]
    wnx, wna, wnu = w1n[:dx], w1n[dx:dx + EOUT], w1n[dx + EOUT:]
    w1g = pg["w1"]
    nout = pn["w2"].shape[1]
    wgm, wgu = w1g[:nout], w1g[nout:]
    xn, gu = _node_mlp(
        x, agg[0], agg[1], u,
        wnx, wna, wnu, pn["b1"].reshape(1, -1), pn["w2"], pn["b2"].reshape(1, -1),
        wgm, wgu, pg["b1"].reshape(1, -1), pg["w2"], pg["b2"].reshape(1, -1),
        n_real)
    return xn, e, gu


def kernel(x, edge_attr, global_attr, params, edge_index):
    row2, col2 = edge_index[0], edge_index[1]
    col = col2
    n_real = x.shape[0]
    n_pad = ((n_real + 127) // 128) * 128
    xp = jnp.pad(x, ((0, n_pad - n_real), (0, 0)))
    xc, ec, uc = _meta_step(params["encoder"], xp, edge_attr, global_attr,
                            row2, col2, col, n_real)
    x0, e0, u0 = xc, ec, uc
    for _ in range(NUM_CORE_STEPS):
        xc, ec, uc = _meta_step(
            params["core"],
            jnp.concatenate([x0, xc], axis=1),
            jnp.concatenate([e0, ec], axis=1),
            jnp.concatenate([u0, uc], axis=1),
            row2, col2, col, n_real)
    xn, e, gu = _meta_step(params["decoder"], xc, ec, uc, row2, col2, col,
                           n_real)
    return xn[:n_real], e, gu


# speedup vs baseline: 3.1422x; 1.0646x over previous
import functools

import jax
import jax.numpy as jnp
from jax import lax
from jax.experimental import pallas as pl
from jax.experimental.pallas import tpu as pltpu
from jax.experimental.pallas import tpu_sc as plsc

F32 = jnp.float32
LAT = 128
EOUT = 16
NUM_CORE_STEPS = 5

NCORES = 2
NSUB = 16
NWORK = NCORES * NSUB
CH = 80
CHS = 2000


def _node_proj_body(x_ref, wsrc_ref, wdst_ref, u_ref, wu_ref, b1_ref,
                    xs_ref, xd_ref):
    x = x_ref[...]
    ub = jnp.dot(u_ref[...], wu_ref[...], preferred_element_type=F32) + b1_ref[...]
    xs_ref[...] = jnp.dot(x, wsrc_ref[...], preferred_element_type=F32) + ub
    xd_ref[...] = jnp.dot(x, wdst_ref[...], preferred_element_type=F32)


def _node_proj(x, wsrc, wdst, u, wu, b1):
    n, dx = x.shape
    du = u.shape[1]
    bn = 1264
    grid = n // bn
    return pl.pallas_call(
        _node_proj_body,
        grid=(grid,),
        in_specs=[
            pl.BlockSpec((bn, dx), lambda i: (i, 0)),
            pl.BlockSpec((dx, LAT), lambda i: (0, 0)),
            pl.BlockSpec((dx, LAT), lambda i: (0, 0)),
            pl.BlockSpec((1, du), lambda i: (0, 0)),
            pl.BlockSpec((du, LAT), lambda i: (0, 0)),
            pl.BlockSpec((1, LAT), lambda i: (0, 0)),
        ],
        out_specs=[pl.BlockSpec((bn, LAT), lambda i: (i, 0)),
                   pl.BlockSpec((bn, LAT), lambda i: (i, 0))],
        out_shape=[jax.ShapeDtypeStruct((n, LAT), F32),
                   jax.ShapeDtypeStruct((n, LAT), F32)],
    )(x, wsrc, wdst, u, wu, b1)


def _edge_mlp_body(gr_ref, gc_ref, ef_ref, wef_ref, w2_ref, b2_ref,
                   e_ref, et_ref):
    h = (gr_ref[...].astype(F32) + gc_ref[...].astype(F32)
         + jnp.dot(ef_ref[...], wef_ref[...], preferred_element_type=F32))
    h = jnp.maximum(h, 0.0)
    e = jnp.dot(h, w2_ref[...], preferred_element_type=F32) + b2_ref[...]
    e_ref[...] = e
    et_ref[...] = jnp.transpose(e)[None]


def _edge_mlp(gr, gc, efeat, wef, w2, b2):
    e_num, de = efeat.shape
    be = CHS
    grid = e_num // be
    return pl.pallas_call(
        _edge_mlp_body,
        grid=(grid,),
        in_specs=[
            pl.BlockSpec((be, LAT), lambda i: (i, 0)),
            pl.BlockSpec((be, LAT), lambda i: (i, 0)),
            pl.BlockSpec((be, de), lambda i: (i, 0)),
            pl.BlockSpec((de, LAT), lambda i: (0, 0)),
            pl.BlockSpec((LAT, EOUT), lambda i: (0, 0)),
            pl.BlockSpec((1, EOUT), lambda i: (0, 0)),
        ],
        out_specs=[pl.BlockSpec((be, EOUT), lambda i: (i, 0)),
                   pl.BlockSpec((1, EOUT, be), lambda i: (i, 0, 0))],
        out_shape=[jax.ShapeDtypeStruct((e_num, EOUT), F32),
                   jax.ShapeDtypeStruct((e_num // be, EOUT, be), F32)],
    )(gr, gc, efeat, wef, w2, b2)


def _node_mlp_body(n_real, x_ref, a0_ref, a1_ref, u_ref,
                   wnx_ref, wna_ref, wnu_ref, b1n_ref, w2n_ref, b2n_ref,
                   wgm_ref, wgu_ref, b1g_ref, w2g_ref, b2g_ref,
                   xn_ref, gu_ref):
    at = a0_ref[...] + a1_ref[...]
    pre = (jnp.dot(x_ref[...], wnx_ref[...], preferred_element_type=F32)
           + jax.lax.dot_general(at, wna_ref[...], (((0,), (0,)), ((), ())),
                                 preferred_element_type=F32)
           + jnp.dot(u_ref[...], wnu_ref[...], preferred_element_type=F32)
           + b1n_ref[...])
    xn = (jnp.dot(jnp.maximum(pre, 0.0), w2n_ref[...],
                  preferred_element_type=F32) + b2n_ref[...])
    rows = jax.lax.broadcasted_iota(jnp.int32, xn.shape, 0)
    xn = jnp.where(rows < n_real, xn, 0.0)
    xn_ref[...] = xn
    mean = jnp.sum(xn, axis=0, keepdims=True) * (1.0 / n_real)
    hg = jnp.maximum(
        jnp.dot(mean, wgm_ref[...], preferred_element_type=F32)
        + jnp.dot(u_ref[...], wgu_ref[...], preferred_element_type=F32)
        + b1g_ref[...], 0.0)
    gu_ref[...] = (jnp.dot(hg, w2g_ref[...], preferred_element_type=F32)
                   + b2g_ref[...])


def _node_mlp(x, a0, a1, u, wnx, wna, wnu, b1n, w2n, b2n,
              wgm, wgu, b1g, w2g, b2g, n_real):
    n, dx = x.shape
    du = u.shape[1]
    dn = w2n.shape[1]
    dg = w2g.shape[1]
    return pl.pallas_call(
        functools.partial(_node_mlp_body, n_real),
        out_shape=[jax.ShapeDtypeStruct((n, dn), F32),
                   jax.ShapeDtypeStruct((1, dg), F32)],
    )(x, a0, a1, u, wnx, wna, wnu, b1n, w2n, b2n, wgm, wgu, b1g, w2g, b2g)


def _edge_gather(xs, xd, row2, col2):
    wds = LAT
    e_num = row2.shape[0]
    ew = e_num // NWORK
    nsc = 5
    bat = nsc * CH
    nbt = ew // bat
    mesh = plsc.VectorSubcoreMesh(core_axis_name="c", subcore_axis_name="s")

    @functools.partial(
        pl.kernel, mesh=mesh,
        out_type=[jax.ShapeDtypeStruct((e_num, wds), F32),
                  jax.ShapeDtypeStruct((e_num, wds), F32)],
        scratch_types=[
            pltpu.VMEM((bat,), jnp.int32), pltpu.VMEM((bat,), jnp.int32),
            pltpu.VMEM((bat, wds), F32), pltpu.VMEM((bat, wds), F32),
            pltpu.SemaphoreType.DMA, pltpu.SemaphoreType.DMA,
        ],
    )
    def gather_k(xs_hbm, xd_hbm, row_hbm, col_hbm, gr_hbm, gc_hbm,
                 ridx, cidx, bufr, bufc, sem_r, sem_c):
        wid = lax.axis_index("s") * NCORES + lax.axis_index("c")

        def body(j, carry):
            base = wid * ew + j * bat
            pltpu.sync_copy(row_hbm.at[pl.ds(base, bat)], ridx)
            pltpu.sync_copy(col_hbm.at[pl.ds(base, bat)], cidx)
            cps = []
            for k in range(nsc):
                cps.append(pltpu.async_copy(
                    xs_hbm.at[ridx.at[pl.ds(k * CH, CH)]],
                    bufr.at[pl.ds(k * CH, CH)], sem_r))
                cps.append(pltpu.async_copy(
                    xd_hbm.at[cidx.at[pl.ds(k * CH, CH)]],
                    bufc.at[pl.ds(k * CH, CH)], sem_c))
            for cp in cps:
                cp.wait()
            ebase = wid * ew + j * bat
            pltpu.sync_copy(bufr, gr_hbm.at[pl.ds(ebase, bat)])
            pltpu.sync_copy(bufc, gc_hbm.at[pl.ds(ebase, bat)])
            return carry

        lax.fori_loop(0, nbt, body, 0)

    return gather_k(xs, xd, row2, col2)


def _edge_scatter(etc, col, n_pad):
    e_num = col.shape[0]
    fph = EOUT // 2
    acc_sz = fph * n_pad
    ec = e_num // NCORES
    ngrp = NSUB // 2
    ew2 = ec // ngrp
    nck = ew2 // CHS
    mesh = plsc.VectorSubcoreMesh(core_axis_name="c", subcore_axis_name="s")

    @functools.partial(
        pl.kernel, mesh=mesh,
        out_type=jax.ShapeDtypeStruct((NCORES * EOUT * n_pad,), F32),
        compiler_params=pltpu.CompilerParams(needs_layout_passes=False,
                                             use_tc_tiling_on_sc=False),
        scratch_types=[
            pltpu.VMEM((CHS,), jnp.int32),
            pltpu.VMEM((fph, CHS), F32),
            pltpu.VMEM((fph, n_pad), F32),
            pltpu.HBM((NCORES, NSUB, acc_sz), F32),
        ],
    )
    def scatter_k(etc_hbm, col_hbm, out_hbm, cidx, ebt, acc, stage):
        cid = lax.axis_index("c")
        sid = lax.axis_index("s")
        fh = sid // ngrp
        grp = sid % ngrp

        def zero_body(k, carry):
            for f in range(fph):
                acc[f, pl.ds(k * 16, 16)] = jnp.zeros((16,), F32)
            return carry

        lax.fori_loop(0, n_pad // 16, zero_body, 0)

        def chunk_body(c, carry):
            ebase = cid * ec + grp * ew2 + c * CHS
            pltpu.sync_copy(col_hbm.at[pl.ds(ebase, CHS)], cidx)
            pltpu.sync_copy(etc_hbm.at[ebase // CHS, pl.ds(fh * fph, fph)],
                            ebt)

            def j_body(j, carry2):
                idx = cidx[pl.ds(j * 16, 16)]
                for f in range(fph):
                    vals = ebt[f, pl.ds(j * 16, 16)]
                    frow = jnp.full((16,), f, jnp.int32)
                    plsc.addupdate_scatter(acc, [frow, idx], vals)
                return carry2

            lax.fori_loop(0, CHS // 16, j_body, 0)
            return carry

        lax.fori_loop(0, nck, chunk_body, 0)

        for f in range(fph):
            pltpu.sync_copy(acc.at[f],
                            stage.at[cid, sid, pl.ds(f * n_pad, n_pad)])
        plsc.subcore_barrier()

        f_m = sid % fph
        fh_m = sid // fph

        def load_body(g, carry):
            pltpu.sync_copy(
                stage.at[cid, fh_m * ngrp + g, pl.ds(f_m * n_pad, n_pad)],
                acc.at[g])
            return carry

        lax.fori_loop(0, ngrp, load_body, 0)

        def red_body(k, carry):
            v = acc[0, pl.ds(k * 16, 16)]
            for g in range(1, ngrp):
                v = v + acc[g, pl.ds(k * 16, 16)]
            acc[0, pl.ds(k * 16, 16)] = v
            return carry

        lax.fori_loop(0, n_pad // 16, red_body, 0)
        pltpu.sync_copy(
            acc.at[0],
            out_hbm.at[pl.ds(cid * EOUT * n_pad + (fh_m * fph + f_m) * n_pad,
                             n_pad)])

    return scatter_k(etc, col)


def _meta_step(p, x, efeat, u, row2, col2, col, n_real):
    n = x.shape[0]
    dx = x.shape[1]
    de = efeat.shape[1]
    pe, pn, pg = p["edge"], p["node"], p["global"]

    w1e = pe["w1"]
    wsrc, wdst = w1e[:dx], w1e[dx:2 * dx]
    wef, wue = w1e[2 * dx:2 * dx + de], w1e[2 * dx + de:]
    xs, xd = _node_proj(x, wsrc, wdst, u, wue, pe["b1"].reshape(1, -1))

    gr, gc = _edge_gather(xs, xd, row2, col2)
    e, etc = _edge_mlp(gr, gc, efeat, wef, pe["w2"], pe["b2"].reshape(1, -1))

    agg = _edge_scatter(etc, col, n).reshape(NCORES, EOUT, n)

    w1n = pn["w1"]
    wnx, wna, wnu = w1n[:dx], w1n[dx:dx + EOUT], w1n[dx + EOUT:]
    w1g = pg["w1"]
    nout = pn["w2"].shape[1]
    wgm, wgu = w1g[:nout], w1g[nout:]
    xn, gu = _node_mlp(
        x, agg[0], agg[1], u,
        wnx, wna, wnu, pn["b1"].reshape(1, -1), pn["w2"], pn["b2"].reshape(1, -1),
        wgm, wgu, pg["b1"].reshape(1, -1), pg["w2"], pg["b2"].reshape(1, -1),
        n_real)
    return xn, e, gu


def kernel(x, edge_attr, global_attr, params, edge_index):
    row2, col2 = edge_index[0], edge_index[1]
    col = col2
    n_real = x.shape[0]
    n_pad = ((n_real + 127) // 128) * 128
    xp = jnp.pad(x, ((0, n_pad - n_real), (0, 0)))
    xc, ec, uc = _meta_step(params["encoder"], xp, edge_attr, global_attr,
                            row2, col2, col, n_real)
    x0, e0, u0 = xc, ec, uc
    for _ in range(NUM_CORE_STEPS):
        xc, ec, uc = _meta_step(
            params["core"],
            jnp.concatenate([x0, xc], axis=1),
            jnp.concatenate([e0, ec], axis=1),
            jnp.concatenate([u0, uc], axis=1),
            row2, col2, col, n_real)
    xn, e, gu = _meta_step(params["decoder"], xc, ec, uc, row2, col2, col,
                           n_real)
    return xn[:n_real], e, gu
